# baseline (speedup 1.0000x reference)
import functools

import jax
import jax.numpy as jnp
from jax.experimental import pallas as pl
from jax.experimental.pallas import tpu as pltpu
from jax.experimental.pallas import tpu_sc as plsc

N_CLASS = 8


def _sc_stream_copy(x_flat):
    total_rows, d = x_flat.shape
    nw = 32
    per_w = total_rows // nw
    batch = 32
    mesh = plsc.VectorSubcoreMesh(core_axis_name="c", subcore_axis_name="s")

    @functools.partial(
        pl.kernel, mesh=mesh,
        out_type=jax.ShapeDtypeStruct((total_rows, d), jnp.float32),
        scratch_types=[pltpu.VMEM((batch, d), jnp.float32)],
    )
    def k(x_hbm, out_hbm, buf):
        wid = jax.lax.axis_index("s") * 2 + jax.lax.axis_index("c")
        base = wid * per_w

        def body(i, carry):
            off = base + i * batch
            pltpu.sync_copy(x_hbm.at[pl.ds(off, batch)], buf)
            pltpu.sync_copy(buf, out_hbm.at[pl.ds(off, batch)])
            return carry

        jax.lax.fori_loop(0, per_w // batch, body, 0)

    return k(x_flat)
_SUB = 32
_D = 256


def _fused_body(idx_ref, x_ref, w_ref, out_ref, p_ref, ys_ref):
    f32 = jnp.float32
    bf16 = jnp.bfloat16
    rows = out_ref.shape[0]
    nsub = rows // _SUB
    ii = jax.lax.broadcasted_iota(jnp.int32, (256, 256), 0)
    jj = jax.lax.broadcasted_iota(jnp.int32, (256, 256), 1)
    band = ii >> 5
    diag = (ii & 31) == (jj >> 3)
    ones_b = jnp.ones((256, 128), dtype=bf16)

    for s in range(nsub):
        idxrow = jnp.broadcast_to(idx_ref[s:s + 1, :], (256, 256))
        hit = diag & (idxrow == band)
        a1 = jnp.where(hit, 1.0, 0.0).astype(bf16)
        cnt = jax.lax.dot_general(a1, ones_b, (((1,), (0,)), ((), ())),
                                  preferred_element_type=f32)
        rc = 1.0 / jnp.maximum(cnt[:, 0:1], 1.0)
        a = jnp.where(hit, rc, 0.0).astype(bf16)
        xs = x_ref[s * 256:(s + 1) * 256, :]
        xh = xs.astype(bf16)
        ps = jax.lax.dot_general(a, xh, (((1,), (0,)), ((), ())),
                                 preferred_element_type=f32)
        p_ref[:, s * _SUB:(s + 1) * _SUB, :] = ps.reshape(N_CLASS, _SUB, _D)

    for c in range(N_CLASS):
        y = jax.lax.dot_general(p_ref[c], w_ref[c],
                                (((1,), (1,)), ((), ())),
                                preferred_element_type=f32)
        ys_ref[c] = jnp.tanh(y)

    ch = 64
    for t in range(rows // ch):
        blk = ys_ref[:, t * ch:(t + 1) * ch, :]
        out_ref[t * ch:(t + 1) * ch, :, :] = jnp.swapaxes(blk, 0, 1)


@functools.partial(jax.jit, static_argnames=("rows",))
def _run(idx_flat, x_flat, w, rows=1024):
    total = x_flat.shape[0] // 8
    grid = (total // rows,)
    nsub = rows // _SUB
    return pl.pallas_call(
        _fused_body,
        grid=grid,
        in_specs=[
            pl.BlockSpec((nsub, 256), lambda i: (i, 0)),
            pl.BlockSpec((rows * 8, _D), lambda i: (i, 0)),
            pl.BlockSpec((N_CLASS, _D, _D), lambda i: (0, 0, 0)),
        ],
        out_specs=pl.BlockSpec((rows, N_CLASS, _D), lambda i: (i, 0, 0)),
        out_shape=jax.ShapeDtypeStruct((total, N_CLASS, _D), jnp.float32),
        scratch_shapes=[
            pltpu.VMEM((N_CLASS, rows, _D), jnp.float32),
            pltpu.VMEM((N_CLASS, rows, _D), jnp.float32),
        ],
        compiler_params=pltpu.CompilerParams(
            dimension_semantics=("arbitrary",),
        ),
    )(idx_flat, x_flat, w)


def kernel(clustered_index_topk, weightedDinput_topk, W):
    b, n, k = clustered_index_topk.shape
    d = weightedDinput_topk.shape[-1]
    total = b * n
    idx_flat = clustered_index_topk.reshape(total // _SUB, _SUB * k)
    x_flat = weightedDinput_topk.reshape(total * k, d)
    x_flat = _sc_stream_copy(x_flat)
    out = _run(idx_flat, x_flat, W)
    return out.reshape(b, n, N_CLASS, W.shape[1])


# speedup vs baseline: 2.7416x; 2.7416x over previous
import functools

import jax
import jax.numpy as jnp
from jax.experimental import pallas as pl
from jax.experimental.pallas import tpu as pltpu

N_CLASS = 8
_SUB = 32
_D = 256


def _fused_body(idx_ref, x_ref, w_ref, out_ref, p_ref, ys_ref):
    f32 = jnp.float32
    bf16 = jnp.bfloat16
    rows = out_ref.shape[0]
    nsub = rows // _SUB
    ii = jax.lax.broadcasted_iota(jnp.int32, (256, 256), 0)
    jj = jax.lax.broadcasted_iota(jnp.int32, (256, 256), 1)
    band = ii >> 5
    diag = (ii & 31) == (jj >> 3)
    ones_b = jnp.ones((256, 128), dtype=bf16)

    for s in range(nsub):
        idxrow = jnp.broadcast_to(idx_ref[s:s + 1, :], (256, 256))
        hit = diag & (idxrow == band)
        a = jnp.where(hit, 1.0, 0.0).astype(bf16)
        cnt = jax.lax.dot_general(a, ones_b, (((1,), (0,)), ((), ())),
                                  preferred_element_type=f32)
        rc = 1.0 / jnp.maximum(cnt[:, 0:1], 1.0)
        xs = x_ref[s * 256:(s + 1) * 256, :]
        xh = xs.astype(bf16)
        ps = jax.lax.dot_general(a, xh, (((1,), (0,)), ((), ())),
                                 preferred_element_type=f32) * rc
        p_ref[:, s * _SUB:(s + 1) * _SUB, :] = ps.reshape(N_CLASS, _SUB, _D)

    for c in range(N_CLASS):
        y = jax.lax.dot_general(p_ref[c], w_ref[c],
                                (((1,), (1,)), ((), ())),
                                preferred_element_type=f32)
        ys_ref[c] = jnp.tanh(y)

    ch = 64
    for t in range(rows // ch):
        blk = ys_ref[:, t * ch:(t + 1) * ch, :]
        out_ref[t * ch:(t + 1) * ch, :, :] = jnp.swapaxes(blk, 0, 1)


@functools.partial(jax.jit, static_argnames=("rows",))
def _run(idx_flat, x_flat, w, rows=1024):
    total = x_flat.shape[0] // 8
    grid = (total // rows,)
    nsub = rows // _SUB
    return pl.pallas_call(
        _fused_body,
        grid=grid,
        in_specs=[
            pl.BlockSpec((nsub, 256), lambda i: (i, 0)),
            pl.BlockSpec((rows * 8, _D), lambda i: (i, 0)),
            pl.BlockSpec((N_CLASS, _D, _D), lambda i: (0, 0, 0)),
        ],
        out_specs=pl.BlockSpec((rows, N_CLASS, _D), lambda i: (i, 0, 0)),
        out_shape=jax.ShapeDtypeStruct((total, N_CLASS, _D), jnp.float32),
        scratch_shapes=[
            pltpu.VMEM((N_CLASS, rows, _D), jnp.float32),
            pltpu.VMEM((N_CLASS, rows, _D), jnp.float32),
        ],
        compiler_params=pltpu.CompilerParams(
            dimension_semantics=("arbitrary",),
        ),
    )(idx_flat, x_flat, w)


def kernel(clustered_index_topk, weightedDinput_topk, W):
    b, n, k = clustered_index_topk.shape
    d = weightedDinput_topk.shape[-1]
    total = b * n
    idx_flat = clustered_index_topk.reshape(total // _SUB, _SUB * k)
    x_flat = weightedDinput_topk.reshape(total * k, d)
    out = _run(idx_flat, x_flat, W)
    return out.reshape(b, n, N_CLASS, W.shape[1])


# speedup vs baseline: 2.7851x; 1.0159x over previous
import functools

import jax
import jax.numpy as jnp
from jax.experimental import pallas as pl
from jax.experimental.pallas import tpu as pltpu

N_CLASS = 8
_SUB = 32
_D = 256


def _fused_body(idx_ref, x_ref, w_ref, out_ref, p_ref, ys_ref):
    f32 = jnp.float32
    bf16 = jnp.bfloat16
    rows = out_ref.shape[0]
    nsub = rows // _SUB
    ii = jax.lax.broadcasted_iota(jnp.int32, (256, 256), 0)
    jj = jax.lax.broadcasted_iota(jnp.int32, (256, 256), 1)
    band = ii >> 5
    diag = (ii & 31) == (jj >> 3)
    ones_b = jnp.ones((256, 128), dtype=bf16)

    for s in range(nsub):
        idxrow = jnp.broadcast_to(idx_ref[s:s + 1, :], (256, 256))
        hit = diag & (idxrow == band)
        a1 = jnp.where(hit, 1.0, 0.0).astype(bf16)
        cnt = jax.lax.dot_general(a1, ones_b, (((1,), (0,)), ((), ())),
                                  preferred_element_type=f32)
        rc = 1.0 / jnp.maximum(cnt[:, 0:1], 1.0)
        a = jnp.where(hit, rc, 0.0).astype(bf16)
        xs = x_ref[s * 256:(s + 1) * 256, :]
        xh = xs.astype(bf16)
        ps = jax.lax.dot_general(a, xh, (((1,), (0,)), ((), ())),
                                 preferred_element_type=f32)
        p_ref[:, s * _SUB:(s + 1) * _SUB, :] = ps.reshape(N_CLASS, _SUB, _D)

    for c in range(N_CLASS):
        y = jax.lax.dot_general(p_ref[c], w_ref[c],
                                (((1,), (1,)), ((), ())),
                                preferred_element_type=f32)
        ys_ref[c] = jnp.tanh(y)

    ch = 64
    for t in range(rows // ch):
        blk = ys_ref[:, t * ch:(t + 1) * ch, :]
        out_ref[t * ch:(t + 1) * ch, :, :] = jnp.swapaxes(blk, 0, 1)


@functools.partial(jax.jit, static_argnames=("rows",))
def _run(idx_flat, x_flat, w, rows=1024):
    total = x_flat.shape[0] // 8
    grid = (total // rows,)
    nsub = rows // _SUB
    return pl.pallas_call(
        _fused_body,
        grid=grid,
        in_specs=[
            pl.BlockSpec((nsub, 256), lambda i: (i, 0)),
            pl.BlockSpec((rows * 8, _D), lambda i: (i, 0)),
            pl.BlockSpec((N_CLASS, _D, _D), lambda i: (0, 0, 0)),
        ],
        out_specs=pl.BlockSpec((rows, N_CLASS, _D), lambda i: (i, 0, 0)),
        out_shape=jax.ShapeDtypeStruct((total, N_CLASS, _D), jnp.float32),
        scratch_shapes=[
            pltpu.VMEM((N_CLASS, rows, _D), jnp.float32),
            pltpu.VMEM((N_CLASS, rows, _D), jnp.float32),
        ],
        compiler_params=pltpu.CompilerParams(
            dimension_semantics=("arbitrary",),
        ),
    )(idx_flat, x_flat, w)


def kernel(clustered_index_topk, weightedDinput_topk, W):
    b, n, k = clustered_index_topk.shape
    d = weightedDinput_topk.shape[-1]
    total = b * n
    idx_flat = clustered_index_topk.reshape(total // _SUB, _SUB * k)
    x_flat = weightedDinput_topk.reshape(total * k, d)
    out = _run(idx_flat, x_flat, W)
    return out.reshape(b, n, N_CLASS, W.shape[1])


# speedup vs baseline: 2.8064x; 1.0076x over previous
import functools

import jax
import jax.numpy as jnp
from jax.experimental import pallas as pl
from jax.experimental.pallas import tpu as pltpu

N_CLASS = 8
_SUB = 32
_D = 256


def _fused_body(idx_ref, x_ref, w_ref, out_ref, p_ref, ys_ref):
    f32 = jnp.float32
    bf16 = jnp.bfloat16
    rows = out_ref.shape[0]
    nsub = rows // _SUB
    ii = jax.lax.broadcasted_iota(jnp.int32, (256, 256), 0)
    comb_i = (ii >> 5) + ((ii & 31) << 3)
    ones_b = jnp.ones((256, 128), dtype=bf16)

    for s in range(nsub):
        idxrow = jnp.broadcast_to(idx_ref[s:s + 1, :], (256, 256))
        hit = idxrow == comb_i
        a1 = jnp.where(hit, 1.0, 0.0).astype(bf16)
        cnt = jax.lax.dot_general(a1, ones_b, (((1,), (0,)), ((), ())),
                                  preferred_element_type=f32)
        rc = 1.0 / jnp.maximum(cnt[:, 0:1], 1.0)
        a = jnp.where(hit, rc, 0.0).astype(bf16)
        xs = x_ref[s * 256:(s + 1) * 256, :]
        xh = xs.astype(bf16)
        ps = jax.lax.dot_general(a, xh, (((1,), (0,)), ((), ())),
                                 preferred_element_type=f32)
        p_ref[:, s * _SUB:(s + 1) * _SUB, :] = ps.reshape(N_CLASS, _SUB, _D)

    for c in range(N_CLASS):
        y = jax.lax.dot_general(p_ref[c], w_ref[c],
                                (((1,), (1,)), ((), ())),
                                preferred_element_type=f32)
        ys_ref[c] = jnp.tanh(y)

    ch = 64
    for t in range(rows // ch):
        blk = ys_ref[:, t * ch:(t + 1) * ch, :]
        out_ref[t * ch:(t + 1) * ch, :, :] = jnp.swapaxes(blk, 0, 1)


@functools.partial(jax.jit, static_argnames=("rows",))
def _run(idx_flat, x_flat, w, rows=1024):
    total = x_flat.shape[0] // 8
    grid = (total // rows,)
    nsub = rows // _SUB
    return pl.pallas_call(
        _fused_body,
        grid=grid,
        in_specs=[
            pl.BlockSpec((nsub, 256), lambda i: (i, 0)),
            pl.BlockSpec((rows * 8, _D), lambda i: (i, 0)),
            pl.BlockSpec((N_CLASS, _D, _D), lambda i: (0, 0, 0)),
        ],
        out_specs=pl.BlockSpec((rows, N_CLASS, _D), lambda i: (i, 0, 0)),
        out_shape=jax.ShapeDtypeStruct((total, N_CLASS, _D), jnp.float32),
        scratch_shapes=[
            pltpu.VMEM((N_CLASS, rows, _D), jnp.float32),
            pltpu.VMEM((N_CLASS, rows, _D), jnp.float32),
        ],
        compiler_params=pltpu.CompilerParams(
            dimension_semantics=("arbitrary",),
        ),
    )(idx_flat, x_flat, w)


def kernel(clustered_index_topk, weightedDinput_topk, W):
    b, n, k = clustered_index_topk.shape
    d = weightedDinput_topk.shape[-1]
    total = b * n
    idx_flat = clustered_index_topk.reshape(total // _SUB, _SUB * k)
    idx_flat = idx_flat + ((jnp.arange(_SUB * k, dtype=jnp.int32) >> 3) << 3)
    x_flat = weightedDinput_topk.reshape(total * k, d)
    out = _run(idx_flat, x_flat, W)
    return out.reshape(b, n, N_CLASS, W.shape[1])
